# baseline (speedup 1.0000x reference)
import jax
import jax.numpy as jnp
from jax import lax
from jax.experimental import pallas as pl
from jax.experimental.pallas import tpu as pltpu

B, S, D, H = 64, 1024, 96, 64
R = 8
HB = 32


def _fused_body(x_ref, w1_ref, b1_ref, w2_ref, b2_ref, tri_ref,
                out_ref, mask_ref, ek_ref,
                xbuf, sbuf, t_sc, need_sc):
    i = pl.program_id(0)

    @pl.when(i < 8)
    def _phase_a():
        x = x_ref[...]
        xbuf[pl.ds(i * R, R), :, :] = x
        x2 = x.reshape(R * S, D)
        h = jnp.dot(x2.astype(jnp.bfloat16), w1_ref[...].astype(jnp.bfloat16),
                    preferred_element_type=jnp.float32)
        h = jnp.maximum(h + b1_ref[...][None, :], 0.0)
        lg = jnp.dot(h.astype(jnp.bfloat16), w2_ref[...].astype(jnp.bfloat16),
                     preferred_element_type=jnp.float32)
        logits = lg.reshape(R, S) + b2_ref[0]

        ek = jnp.sum(jax.nn.sigmoid(logits), axis=1, keepdims=True)
        ek_ref[pl.ds(i * R, R), :] = ek

        m = jnp.max(logits, axis=1, keepdims=True)
        e = jnp.exp(logits - m)
        sbuf[pl.ds(i * R, R), :] = e / jnp.sum(e, axis=1, keepdims=True)

    def _search(lo):
        s = sbuf[pl.ds(lo, HB), :]
        kf = (jnp.maximum(ek_ref[pl.ds(lo, HB), :].astype(jnp.int32), 32)
              .astype(jnp.float32))
        u = lax.bitcast_convert_type(s, jnp.int32)
        ones_col = jnp.ones((S, 1), jnp.bfloat16)

        def count_ge(cand):
            return jnp.dot((u >= cand).astype(jnp.bfloat16), ones_col,
                           preferred_element_type=jnp.float32)

        def body(tt, p):
            cand = p | lax.shift_left(jnp.int32(1), 29 - tt)
            return jnp.where(count_ge(cand) >= kf, cand, p)

        t = lax.fori_loop(0, 30, body, jnp.zeros((HB, 1), jnp.int32))
        t_sc[pl.ds(lo, HB), :] = t
        cnt_gt = jnp.dot((u > t).astype(jnp.bfloat16), ones_col,
                         preferred_element_type=jnp.float32)
        need_sc[pl.ds(lo, HB), :] = kf - cnt_gt

    @pl.when(i == 8)
    def _search_all():
        _search(0)
        _search(HB)

    @pl.when(i >= 8)
    def _phase_b():
        j = i - 8
        s_rows = sbuf[pl.ds(j * R, R), :]
        u_rows = lax.bitcast_convert_type(s_rows, jnp.int32)
        t_rows = t_sc[pl.ds(j * R, R), :]
        need_rows = need_sc[pl.ds(j * R, R), :]
        gt = u_rows > t_rows
        eq = u_rows == t_rows
        pre = jnp.dot(eq.astype(jnp.bfloat16), tri_ref[...],
                      preferred_element_type=jnp.float32)
        sel = gt | (eq & (pre < need_rows))
        hard = sel.astype(jnp.float32)
        sel_mask = (hard - s_rows) + s_rows
        out_ref[...] = xbuf[pl.ds(j * R, R), :, :] * sel_mask[:, :, None]
        mask_ref[...] = sel_mask


@jax.jit
def kernel(token_embeddings, W1, b1, W2, b2):
    tri = jnp.triu(jnp.ones((S, S), jnp.bfloat16), k=1)
    out, mask, ek = pl.pallas_call(
        _fused_body,
        grid=(16,),
        in_specs=[
            pl.BlockSpec((R, S, D), lambda i: (jnp.minimum(i, 7), 0, 0)),
            pl.BlockSpec((D, H), lambda i: (0, 0)),
            pl.BlockSpec((H,), lambda i: (0,)),
            pl.BlockSpec((H, 1), lambda i: (0, 0)),
            pl.BlockSpec((1,), lambda i: (0,)),
            pl.BlockSpec((S, S), lambda i: (0, 0)),
        ],
        out_specs=[
            pl.BlockSpec((R, S, D), lambda i: (jnp.maximum(i - 8, 0), 0, 0)),
            pl.BlockSpec((R, S), lambda i: (jnp.maximum(i - 8, 0), 0)),
            pl.BlockSpec((B, 1), lambda i: (0, 0)),
        ],
        out_shape=[
            jax.ShapeDtypeStruct((B, S, D), jnp.float32),
            jax.ShapeDtypeStruct((B, S), jnp.float32),
            jax.ShapeDtypeStruct((B, 1), jnp.float32),
        ],
        scratch_shapes=[
            pltpu.VMEM((B, S, D), jnp.float32),
            pltpu.VMEM((B, S), jnp.float32),
            pltpu.VMEM((B, 1), jnp.int32),
            pltpu.VMEM((B, 1), jnp.float32),
        ],
    )(token_embeddings, W1, b1, W2, b2, tri)
    return out, mask, ek[:, 0]


# speedup vs baseline: 1.0272x; 1.0272x over previous
import jax
import jax.numpy as jnp
from jax import lax
from jax.experimental import pallas as pl
from jax.experimental.pallas import tpu as pltpu

B, S, D, H = 64, 1024, 96, 64
R = 8
HB = 32


def _fused_body(x_ref, w1_ref, b1_ref, w2_ref, b2_ref, tri_ref,
                out_ref, mask_ref, ek_ref,
                xbuf, sbuf, t_sc, need_sc):
    i = pl.program_id(0)

    @pl.when(i < 8)
    def _phase_a():
        x = x_ref[...]
        xbuf[pl.ds(i * R, R), :, :] = x
        x2 = x.reshape(R * S, D)
        h = jnp.dot(x2.astype(jnp.bfloat16), w1_ref[...].astype(jnp.bfloat16),
                    preferred_element_type=jnp.float32)
        h = jnp.maximum(h + b1_ref[...][None, :], 0.0)
        h3 = h.reshape(R, S, H).astype(jnp.bfloat16).astype(jnp.float32)
        w2 = (w2_ref[...].reshape(1, 1, H)
              .astype(jnp.bfloat16).astype(jnp.float32))
        logits = jnp.sum(h3 * w2, axis=2) + b2_ref[0]

        ek = jnp.sum(jax.nn.sigmoid(logits), axis=1, keepdims=True)
        ek_ref[pl.ds(i * R, R), :] = ek

        m = jnp.max(logits, axis=1, keepdims=True)
        e = jnp.exp(logits - m)
        sbuf[pl.ds(i * R, R), :] = e / jnp.sum(e, axis=1, keepdims=True)

    def _search(lo):
        s = sbuf[pl.ds(lo, HB), :]
        kf = (jnp.maximum(ek_ref[pl.ds(lo, HB), :].astype(jnp.int32), 32)
              .astype(jnp.float32))
        u = lax.bitcast_convert_type(s, jnp.int32)
        ones_col = jnp.ones((S, 1), jnp.bfloat16)

        def count_ge(cand):
            return jnp.dot((u >= cand).astype(jnp.bfloat16), ones_col,
                           preferred_element_type=jnp.float32)

        def body(tt, p):
            cand = p | lax.shift_left(jnp.int32(1), 29 - tt)
            return jnp.where(count_ge(cand) >= kf, cand, p)

        t = lax.fori_loop(0, 30, body, jnp.zeros((HB, 1), jnp.int32))
        t_sc[pl.ds(lo, HB), :] = t
        cnt_gt = jnp.dot((u > t).astype(jnp.bfloat16), ones_col,
                         preferred_element_type=jnp.float32)
        need_sc[pl.ds(lo, HB), :] = kf - cnt_gt

    @pl.when(i == 8)
    def _search_all():
        _search(0)
        _search(HB)

    @pl.when(i >= 8)
    def _phase_b():
        j = i - 8
        s_rows = sbuf[pl.ds(j * R, R), :]
        u_rows = lax.bitcast_convert_type(s_rows, jnp.int32)
        t_rows = t_sc[pl.ds(j * R, R), :]
        need_rows = need_sc[pl.ds(j * R, R), :]
        gt = u_rows > t_rows
        eq = u_rows == t_rows
        pre = jnp.dot(eq.astype(jnp.bfloat16), tri_ref[...],
                      preferred_element_type=jnp.float32)
        sel = gt | (eq & (pre < need_rows))
        hard = sel.astype(jnp.float32)
        sel_mask = (hard - s_rows) + s_rows
        out_ref[...] = xbuf[pl.ds(j * R, R), :, :] * sel_mask[:, :, None]
        mask_ref[...] = sel_mask


@jax.jit
def kernel(token_embeddings, W1, b1, W2, b2):
    tri = jnp.triu(jnp.ones((S, S), jnp.bfloat16), k=1)
    out, mask, ek = pl.pallas_call(
        _fused_body,
        grid=(16,),
        in_specs=[
            pl.BlockSpec((R, S, D), lambda i: (jnp.minimum(i, 7), 0, 0)),
            pl.BlockSpec((D, H), lambda i: (0, 0)),
            pl.BlockSpec((H,), lambda i: (0,)),
            pl.BlockSpec((H, 1), lambda i: (0, 0)),
            pl.BlockSpec((1,), lambda i: (0,)),
            pl.BlockSpec((S, S), lambda i: (0, 0)),
        ],
        out_specs=[
            pl.BlockSpec((R, S, D), lambda i: (jnp.maximum(i - 8, 0), 0, 0)),
            pl.BlockSpec((R, S), lambda i: (jnp.maximum(i - 8, 0), 0)),
            pl.BlockSpec((B, 1), lambda i: (0, 0)),
        ],
        out_shape=[
            jax.ShapeDtypeStruct((B, S, D), jnp.float32),
            jax.ShapeDtypeStruct((B, S), jnp.float32),
            jax.ShapeDtypeStruct((B, 1), jnp.float32),
        ],
        scratch_shapes=[
            pltpu.VMEM((B, S, D), jnp.float32),
            pltpu.VMEM((B, S), jnp.float32),
            pltpu.VMEM((B, 1), jnp.int32),
            pltpu.VMEM((B, 1), jnp.float32),
        ],
    )(token_embeddings, W1, b1, W2, b2, tri)
    return out, mask, ek[:, 0]


# speedup vs baseline: 1.0829x; 1.0542x over previous
import jax
import jax.numpy as jnp
from jax import lax
from jax.experimental import pallas as pl
from jax.experimental.pallas import tpu as pltpu

B, S, D, H = 64, 1024, 96, 64
R = 8
HB = 64


def _fused_body(x_ref, w1_ref, b1_ref, w2_ref, b2_ref, tri_ref,
                out_ref, mask_ref, ek_ref,
                xbuf, sbuf, t_sc, need_sc):
    i = pl.program_id(0)

    @pl.when(i < 8)
    def _phase_a():
        x = x_ref[...]
        xbuf[pl.ds(i * R, R), :, :] = x
        x2 = x.reshape(R * S, D)
        h = jnp.dot(x2.astype(jnp.bfloat16), w1_ref[...].astype(jnp.bfloat16),
                    preferred_element_type=jnp.float32)
        h = jnp.maximum(h + b1_ref[...][None, :], 0.0)
        h3 = h.reshape(R, S, H).astype(jnp.bfloat16).astype(jnp.float32)
        w2 = (w2_ref[...].reshape(1, 1, H)
              .astype(jnp.bfloat16).astype(jnp.float32))
        logits = jnp.sum(h3 * w2, axis=2) + b2_ref[0]

        ek = jnp.sum(jax.nn.sigmoid(logits), axis=1, keepdims=True)
        ek_ref[pl.ds(i * R, R), :] = ek

        m = jnp.max(logits, axis=1, keepdims=True)
        e = jnp.exp(logits - m)
        sbuf[pl.ds(i * R, R), :] = e / jnp.sum(e, axis=1, keepdims=True)

    def _search(lo):
        s = sbuf[pl.ds(lo, HB), :]
        kf = (jnp.maximum(ek_ref[pl.ds(lo, HB), :].astype(jnp.int32), 32)
              .astype(jnp.float32))
        u = lax.bitcast_convert_type(s, jnp.int32)
        ones_col = jnp.ones((S, 1), jnp.bfloat16)

        def count_ge(cand):
            return jnp.dot((u >= cand).astype(jnp.bfloat16), ones_col,
                           preferred_element_type=jnp.float32)

        def body(tt, p):
            cand = p | lax.shift_left(jnp.int32(1), 29 - tt)
            return jnp.where(count_ge(cand) >= kf, cand, p)

        t = lax.fori_loop(0, 30, body, jnp.zeros((HB, 1), jnp.int32))
        t_sc[pl.ds(lo, HB), :] = t
        cnt_gt = jnp.dot((u > t).astype(jnp.bfloat16), ones_col,
                         preferred_element_type=jnp.float32)
        need_sc[pl.ds(lo, HB), :] = kf - cnt_gt

    @pl.when(i == 8)
    def _search_all():
        _search(0)

    @pl.when(i >= 8)
    def _phase_b():
        j = i - 8
        s_rows = sbuf[pl.ds(j * R, R), :]
        u_rows = lax.bitcast_convert_type(s_rows, jnp.int32)
        t_rows = t_sc[pl.ds(j * R, R), :]
        need_rows = need_sc[pl.ds(j * R, R), :]
        gt = u_rows > t_rows
        eq = u_rows == t_rows
        pre = jnp.dot(eq.astype(jnp.bfloat16), tri_ref[...],
                      preferred_element_type=jnp.float32)
        sel = gt | (eq & (pre < need_rows))
        hard = sel.astype(jnp.float32)
        sel_mask = (hard - s_rows) + s_rows
        out_ref[...] = xbuf[pl.ds(j * R, R), :, :] * sel_mask[:, :, None]
        mask_ref[...] = sel_mask


@jax.jit
def kernel(token_embeddings, W1, b1, W2, b2):
    tri = jnp.triu(jnp.ones((S, S), jnp.bfloat16), k=1)
    out, mask, ek = pl.pallas_call(
        _fused_body,
        grid=(16,),
        in_specs=[
            pl.BlockSpec((R, S, D), lambda i: (jnp.minimum(i, 7), 0, 0)),
            pl.BlockSpec((D, H), lambda i: (0, 0)),
            pl.BlockSpec((H,), lambda i: (0,)),
            pl.BlockSpec((H, 1), lambda i: (0, 0)),
            pl.BlockSpec((1,), lambda i: (0,)),
            pl.BlockSpec((S, S), lambda i: (0, 0)),
        ],
        out_specs=[
            pl.BlockSpec((R, S, D), lambda i: (jnp.maximum(i - 8, 0), 0, 0)),
            pl.BlockSpec((R, S), lambda i: (jnp.maximum(i - 8, 0), 0)),
            pl.BlockSpec((B, 1), lambda i: (0, 0)),
        ],
        out_shape=[
            jax.ShapeDtypeStruct((B, S, D), jnp.float32),
            jax.ShapeDtypeStruct((B, S), jnp.float32),
            jax.ShapeDtypeStruct((B, 1), jnp.float32),
        ],
        scratch_shapes=[
            pltpu.VMEM((B, S, D), jnp.float32),
            pltpu.VMEM((B, S), jnp.float32),
            pltpu.VMEM((B, 1), jnp.int32),
            pltpu.VMEM((B, 1), jnp.float32),
        ],
    )(token_embeddings, W1, b1, W2, b2, tri)
    return out, mask, ek[:, 0]


# speedup vs baseline: 1.0866x; 1.0034x over previous
import jax
import jax.numpy as jnp
from jax import lax
from jax.experimental import pallas as pl
from jax.experimental.pallas import tpu as pltpu

B, S, D, H = 64, 1024, 96, 64
R = 8
HB = 64


def _fused_body(x_ref, w1_ref, b1_ref, w2_ref, b2_ref, tri_ref,
                out_ref, mask_ref, ek_ref,
                xbuf, sbuf, t_sc, need_sc):
    i = pl.program_id(0)

    @pl.when(i < 8)
    def _phase_a():
        x = x_ref[...]
        xbuf[pl.ds(i * R, R), :, :] = x
        x2 = x.reshape(R * S, D)
        h = jnp.dot(x2, w1_ref[...], precision=lax.Precision.DEFAULT,
                    preferred_element_type=jnp.float32)
        h = jnp.maximum(h + b1_ref[...][None, :], 0.0)
        h3 = h.reshape(R, S, H).astype(jnp.bfloat16).astype(jnp.float32)
        w2 = (w2_ref[...].reshape(1, 1, H)
              .astype(jnp.bfloat16).astype(jnp.float32))
        logits = jnp.sum(h3 * w2, axis=2) + b2_ref[0]

        ek = jnp.sum(jax.nn.sigmoid(logits), axis=1, keepdims=True)
        ek_ref[pl.ds(i * R, R), :] = ek

        m = jnp.max(logits, axis=1, keepdims=True)
        e = jnp.exp(logits - m)
        sbuf[pl.ds(i * R, R), :] = e / jnp.sum(e, axis=1, keepdims=True)

    def _search(lo):
        s = sbuf[pl.ds(lo, HB), :]
        kf = (jnp.maximum(ek_ref[pl.ds(lo, HB), :].astype(jnp.int32), 32)
              .astype(jnp.float32))
        u = lax.bitcast_convert_type(s, jnp.int32)
        ones_col = jnp.ones((S, 1), jnp.bfloat16)

        def count_ge(cand):
            return jnp.dot((u >= cand).astype(jnp.bfloat16), ones_col,
                           preferred_element_type=jnp.float32)

        def body(tt, p):
            cand = p | lax.shift_left(jnp.int32(1), 29 - tt)
            return jnp.where(count_ge(cand) >= kf, cand, p)

        t = lax.fori_loop(0, 30, body, jnp.zeros((HB, 1), jnp.int32))
        t_sc[pl.ds(lo, HB), :] = t
        cnt_gt = jnp.dot((u > t).astype(jnp.bfloat16), ones_col,
                         preferred_element_type=jnp.float32)
        need_sc[pl.ds(lo, HB), :] = kf - cnt_gt

    @pl.when(i == 8)
    def _search_all():
        _search(0)

    @pl.when(i >= 8)
    def _phase_b():
        j = i - 8
        s_rows = sbuf[pl.ds(j * R, R), :]
        u_rows = lax.bitcast_convert_type(s_rows, jnp.int32)
        t_rows = t_sc[pl.ds(j * R, R), :]
        need_rows = need_sc[pl.ds(j * R, R), :]
        gt = u_rows > t_rows
        eq = u_rows == t_rows
        pre = jnp.dot(eq.astype(jnp.bfloat16), tri_ref[...],
                      preferred_element_type=jnp.float32)
        sel = gt | (eq & (pre < need_rows))
        hard = sel.astype(jnp.float32)
        sel_mask = (hard - s_rows) + s_rows
        out_ref[...] = xbuf[pl.ds(j * R, R), :, :] * sel_mask[:, :, None]
        mask_ref[...] = sel_mask


@jax.jit
def kernel(token_embeddings, W1, b1, W2, b2):
    tri = jnp.triu(jnp.ones((S, S), jnp.bfloat16), k=1)
    out, mask, ek = pl.pallas_call(
        _fused_body,
        grid=(16,),
        in_specs=[
            pl.BlockSpec((R, S, D), lambda i: (jnp.minimum(i, 7), 0, 0)),
            pl.BlockSpec((D, H), lambda i: (0, 0)),
            pl.BlockSpec((H,), lambda i: (0,)),
            pl.BlockSpec((H, 1), lambda i: (0, 0)),
            pl.BlockSpec((1,), lambda i: (0,)),
            pl.BlockSpec((S, S), lambda i: (0, 0)),
        ],
        out_specs=[
            pl.BlockSpec((R, S, D), lambda i: (jnp.maximum(i - 8, 0), 0, 0)),
            pl.BlockSpec((R, S), lambda i: (jnp.maximum(i - 8, 0), 0)),
            pl.BlockSpec((B, 1), lambda i: (0, 0)),
        ],
        out_shape=[
            jax.ShapeDtypeStruct((B, S, D), jnp.float32),
            jax.ShapeDtypeStruct((B, S), jnp.float32),
            jax.ShapeDtypeStruct((B, 1), jnp.float32),
        ],
        scratch_shapes=[
            pltpu.VMEM((B, S, D), jnp.float32),
            pltpu.VMEM((B, S), jnp.float32),
            pltpu.VMEM((B, 1), jnp.int32),
            pltpu.VMEM((B, 1), jnp.float32),
        ],
    )(token_embeddings, W1, b1, W2, b2, tri)
    return out, mask, ek[:, 0]
